# baseline (speedup 1.0000x reference)
import functools

import jax
import jax.numpy as jnp
from jax import lax
from jax.experimental import pallas as pl
from jax.experimental.pallas import tpu as pltpu
from jax.experimental.pallas import tpu_sc as plsc

N_NODE = 8000
N_ATTRI = 2000
N_TOT = N_NODE + N_ATTRI
E_EDGES = 320000
F = 128
NSUB = 16
CH = 128
NCHT = E_EDGES // CH
CPW = 160
NCHP = CPW * NSUB
SLAB = 16
NSLB = CPW // SLAB
NP = 10240
RPT = NP // NSUB
RB = 128
MROWS = 1000


def _dense_body(x0_ref, w1_ref, wg1_ref, wg2_ref, x1_ref, h_ref):
    x0 = x0_ref[...]
    dn = (((1,), (1,)), ((), ()))
    x1 = lax.dot_general(x0, w1_ref[...], dn, preferred_element_type=jnp.float32)
    x1_ref[...] = x1
    h_ref[0] = lax.dot_general(x1, wg1_ref[...], dn,
                               preferred_element_type=jnp.float32)
    h_ref[1] = lax.dot_general(x1, wg2_ref[...], dn,
                               preferred_element_type=jnp.float32)


_dense = pl.pallas_call(
    _dense_body,
    grid=(N_TOT // MROWS,),
    in_specs=[
        pl.BlockSpec((MROWS, F), lambda i: (i, 0)),
        pl.BlockSpec((F, F), lambda i: (0, 0)),
        pl.BlockSpec((F, F), lambda i: (0, 0)),
        pl.BlockSpec((F, F), lambda i: (0, 0)),
    ],
    out_specs=[
        pl.BlockSpec((MROWS, F), lambda i: (i, 0)),
        pl.BlockSpec((2, MROWS, F), lambda i: (0, i, 0)),
    ],
    out_shape=[
        jax.ShapeDtypeStruct((N_TOT, F), jnp.float32),
        jax.ShapeDtypeStruct((2, N_TOT, F), jnp.float32),
    ],
)


_mesh = plsc.VectorSubcoreMesh(core_axis_name="c", subcore_axis_name="s")


@functools.partial(
    pl.kernel,
    out_type=jax.ShapeDtypeStruct((2, NP, F), jnp.float32),
    mesh=_mesh,
    scratch_types=[
        pltpu.VMEM((CH, F), jnp.float32),
        pltpu.VMEM((CH, F), jnp.float32),
        pltpu.VMEM((SLAB, 2, CH), jnp.int32),
        pltpu.VMEM((SLAB, 2, CH), jnp.int32),
        pltpu.VMEM((SLAB, CH), jnp.float32),
        pltpu.VMEM((SLAB, CH), jnp.float32),
        pltpu.VMEM_SHARED((NP, F), jnp.float32),
        pltpu.SemaphoreType.DMA,
        [pltpu.SemaphoreType.DMA] * 2,
        [pltpu.SemaphoreType.DMA] * 2,
    ],
)
def _sc_spmm(h_st, edges_st, vals_st, out_st,
             rows0, rows1, eslabA, eslabB, vslabA, vslabB, acc, si, sg, ss):
    c = lax.axis_index("c")
    s = lax.axis_index("s")
    rows = (rows0, rows1)
    h = h_st.at[c]
    edges = edges_st.at[c]
    vals = vals_st.at[c]

    gdn = lax.GatherDimensionNumbers(
        offset_dims=(), collapsed_slice_dims=(0,), start_index_map=(0,))

    def _scale(rb, vs, kk):
        def _sbody(g, carry):
            vals16 = vs[kk, pl.ds(g * 16, 16)]
            for i2 in range(16):
                vb = lax.gather(
                    vals16, jnp.full((16, 1), i2, jnp.int32), gdn, (1,),
                    mode=lax.GatherScatterMode.PROMISE_IN_BOUNDS)
                for j in range(F // 16):
                    idx = (g * 16 + i2, pl.ds(j * 16, 16))
                    rows[rb][idx] = rows[rb][idx] * vb
            return carry

        lax.fori_loop(0, CH // 16, _sbody, 0)

    def _gissue(es, kk, rb):
        pltpu.async_copy(h.at[es.at[kk, 1]], rows[rb], sg[rb])

    def _sissue(es, kk, rb):
        pltpu.async_copy(rows[rb], acc.at[es.at[kk, 0]], ss[rb], add=True)

    def _gwait(rb):
        pltpu.make_async_copy(h.at[eslabA.at[0, 1]], rows[rb], sg[rb]).wait()

    def _swait(rb):
        pltpu.make_async_copy(rows[rb], acc.at[eslabA.at[0, 0]], ss[rb]).wait()

    def _iwait(es, vs):
        pltpu.make_async_copy(
            edges.at[pl.ds(s * CPW, SLAB)], es, si).wait()
        pltpu.make_async_copy(
            vals.at[pl.ds(s * CPW, SLAB)], vs, si).wait()

    def _zbody(r, carry):
        for j in range(F // 16):
            rows0[r, pl.ds(j * 16, 16)] = jnp.zeros((16,), jnp.float32)
        return carry

    lax.fori_loop(0, CH, _zbody, 0)
    for k in range(RPT // RB):
        pltpu.sync_copy(rows0, acc.at[pl.ds(s * RPT + k * RB, RB)])
    plsc.subcore_barrier()

    def _slab_body(m, es, vs, eo, vo):
        @pl.when(m > 0)
        def _():
            _swait(0)
            _swait(1)

        _iwait(es, vs)

        @pl.when(m + 1 < NSLB)
        def _():
            nxt = s * CPW + (m + 1) * SLAB
            pltpu.async_copy(edges.at[pl.ds(nxt, SLAB)], eo, si)
            pltpu.async_copy(vals.at[pl.ds(nxt, SLAB)], vo, si)

        _gissue(es, 0, 0)

        def _pair(t, carry):
            kk = 2 * t
            _gwait(0)

            @pl.when(t > 0)
            def _():
                _swait(1)

            _gissue(es, kk + 1, 1)
            _scale(0, vs, kk)
            _sissue(es, kk, 0)
            _gwait(1)

            @pl.when(t < SLAB // 2 - 1)
            def _():
                _swait(0)
                _gissue(es, kk + 2, 0)

            _scale(1, vs, kk + 1)
            _sissue(es, kk + 1, 1)
            return carry

        lax.fori_loop(0, SLAB // 2, _pair, 0)

    pltpu.async_copy(edges.at[pl.ds(s * CPW, SLAB)], eslabA, si)
    pltpu.async_copy(vals.at[pl.ds(s * CPW, SLAB)], vslabA, si)

    def _slabpair(u, carry):
        _slab_body(2 * u, eslabA, vslabA, eslabB, vslabB)
        _slab_body(2 * u + 1, eslabB, vslabB, eslabA, vslabA)
        return carry

    lax.fori_loop(0, NSLB // 2, _slabpair, 0)
    _swait(0)
    _swait(1)
    plsc.subcore_barrier()

    for kb in range(RPT // RB):
        r0 = s * RPT + kb * RB
        pltpu.sync_copy(acc.at[pl.ds(r0, RB)], rows0)

        def _rbody(r, carry):
            for j in range(F // 16):
                rows0[r, pl.ds(j * 16, 16)] = jnp.maximum(
                    rows0[r, pl.ds(j * 16, 16)], 0.0)
            return carry

        lax.fori_loop(0, RB, _rbody, 0)
        pltpu.sync_copy(rows0, out_st.at[c].at[pl.ds(r0, RB)])


def _prep_edges(idx2, val):
    padc = NCHP - NCHT
    dst2 = jnp.pad(idx2[0].reshape(NCHT, CH), ((0, padc), (0, 0)),
                   constant_values=NP - 1)
    src2 = jnp.pad(idx2[1].reshape(NCHT, CH), ((0, padc), (0, 0)))
    val2 = jnp.pad(val.reshape(NCHT, CH), ((0, padc), (0, 0)))
    return jnp.stack([dst2, src2], axis=1), val2


def kernel(adj_indices, adj_values, adj2_indices, adj2_values,
           emb_node, emb_attri, W_trans1, W_gc1, W_gc2):
    x0 = jnp.concatenate([emb_node, emb_attri], axis=0)
    x1, h_st = _dense(x0, W_trans1, W_gc1, W_gc2)
    e1, v1 = _prep_edges(adj_indices, adj_values)
    e2, v2 = _prep_edges(adj2_indices, adj2_values)
    edges_st = jnp.stack([e1, e2])
    vals_st = jnp.stack([v1, v2])
    out = _sc_spmm(h_st, edges_st, vals_st)
    return (x1, out[0, :N_TOT], out[1, :N_TOT])


# speedup vs baseline: 1.0031x; 1.0031x over previous
import functools

import jax
import jax.numpy as jnp
from jax import lax
from jax.experimental import pallas as pl
from jax.experimental.pallas import tpu as pltpu
from jax.experimental.pallas import tpu_sc as plsc

N_NODE = 8000
N_ATTRI = 2000
N_TOT = N_NODE + N_ATTRI
E_EDGES = 320000
F = 128
NSUB = 16
CH = 128
NCHT = E_EDGES // CH
CPW = 160
NCHP = CPW * NSUB
SLAB = 16
NSLB = CPW // SLAB
NP = 10240
RPT = NP // NSUB
RB = 128
MROWS = 1000


def _dense_body(x0_ref, w1_ref, wg1_ref, wg2_ref, x1_ref, h_ref):
    x0 = x0_ref[...]
    dn = (((1,), (1,)), ((), ()))
    x1 = lax.dot_general(x0, w1_ref[...], dn, preferred_element_type=jnp.float32)
    x1_ref[...] = x1
    h_ref[0] = lax.dot_general(x1, wg1_ref[...], dn,
                               preferred_element_type=jnp.float32)
    h_ref[1] = lax.dot_general(x1, wg2_ref[...], dn,
                               preferred_element_type=jnp.float32)


_dense = pl.pallas_call(
    _dense_body,
    grid=(N_TOT // MROWS,),
    in_specs=[
        pl.BlockSpec((MROWS, F), lambda i: (i, 0)),
        pl.BlockSpec((F, F), lambda i: (0, 0)),
        pl.BlockSpec((F, F), lambda i: (0, 0)),
        pl.BlockSpec((F, F), lambda i: (0, 0)),
    ],
    out_specs=[
        pl.BlockSpec((MROWS, F), lambda i: (i, 0)),
        pl.BlockSpec((2, MROWS, F), lambda i: (0, i, 0)),
    ],
    out_shape=[
        jax.ShapeDtypeStruct((N_TOT, F), jnp.float32),
        jax.ShapeDtypeStruct((2, N_TOT, F), jnp.float32),
    ],
)


_mesh = plsc.VectorSubcoreMesh(core_axis_name="c", subcore_axis_name="s")


@functools.partial(
    pl.kernel,
    out_type=jax.ShapeDtypeStruct((2, NP, F), jnp.float32),
    mesh=_mesh,
    scratch_types=[
        pltpu.VMEM((CH, F), jnp.float32),
        pltpu.VMEM((CH, F), jnp.float32),
        pltpu.VMEM((SLAB, 2, CH), jnp.int32),
        pltpu.VMEM((SLAB, 2, CH), jnp.int32),
        pltpu.VMEM((SLAB, CH), jnp.float32),
        pltpu.VMEM((SLAB, CH), jnp.float32),
        pltpu.VMEM_SHARED((NP, F), jnp.float32),
        pltpu.SemaphoreType.DMA,
        [pltpu.SemaphoreType.DMA] * 2,
        [pltpu.SemaphoreType.DMA] * 2,
    ],
)
def _sc_spmm(h_st, edges_st, vals_st, out_st,
             rows0, rows1, eslabA, eslabB, vslabA, vslabB, acc, si, sg, ss):
    c = lax.axis_index("c")
    s = lax.axis_index("s")
    rows = (rows0, rows1)
    h = h_st.at[c]
    edges = edges_st.at[c]
    vals = vals_st.at[c]

    gdn = lax.GatherDimensionNumbers(
        offset_dims=(), collapsed_slice_dims=(0,), start_index_map=(0,))

    def _scale(rb, vs, kk):
        def _sbody(g, carry):
            vals16 = vs[kk, pl.ds(g * 16, 16)]
            for i2 in range(16):
                vb = lax.gather(
                    vals16, jnp.full((16, 1), i2, jnp.int32), gdn, (1,),
                    mode=lax.GatherScatterMode.PROMISE_IN_BOUNDS)
                for j in range(F // 16):
                    idx = (g * 16 + i2, pl.ds(j * 16, 16))
                    rows[rb][idx] = rows[rb][idx] * vb
            return carry

        lax.fori_loop(0, CH // 16, _sbody, 0)

    def _gissue(es, kk, rb):
        pltpu.async_copy(h.at[es.at[kk, 1]], rows[rb], sg[rb])

    def _sissue(es, kk, rb):
        pltpu.async_copy(rows[rb], acc.at[es.at[kk, 0]], ss[rb], add=True)

    def _gwait(rb):
        pltpu.make_async_copy(h.at[eslabA.at[0, 1]], rows[rb], sg[rb]).wait()

    def _swait(rb):
        pltpu.make_async_copy(rows[rb], acc.at[eslabA.at[0, 0]], ss[rb]).wait()

    def _iwait(es, vs):
        pltpu.make_async_copy(
            edges.at[pl.ds(s * CPW, SLAB)], es, si).wait()
        pltpu.make_async_copy(
            vals.at[pl.ds(s * CPW, SLAB)], vs, si).wait()

    def _zbody(r, carry):
        for j in range(F // 16):
            rows0[r, pl.ds(j * 16, 16)] = jnp.zeros((16,), jnp.float32)
        return carry

    lax.fori_loop(0, CH, _zbody, 0)
    for k in range(RPT // RB):
        pltpu.async_copy(rows0, acc.at[pl.ds(s * RPT + k * RB, RB)], sg[0])
    for k in range(RPT // RB):
        pltpu.make_async_copy(rows0, acc.at[pl.ds(s * RPT, RB)], sg[0]).wait()
    plsc.subcore_barrier()

    def _slab_body(m, es, vs, eo, vo):
        @pl.when(m > 0)
        def _():
            _swait(0)
            _swait(1)

        _iwait(es, vs)

        @pl.when(m + 1 < NSLB)
        def _():
            nxt = s * CPW + (m + 1) * SLAB
            pltpu.async_copy(edges.at[pl.ds(nxt, SLAB)], eo, si)
            pltpu.async_copy(vals.at[pl.ds(nxt, SLAB)], vo, si)

        _gissue(es, 0, 0)

        def _pair(t, carry):
            kk = 2 * t
            _gwait(0)

            @pl.when(t > 0)
            def _():
                _swait(1)

            _gissue(es, kk + 1, 1)
            _scale(0, vs, kk)
            _sissue(es, kk, 0)
            _gwait(1)

            @pl.when(t < SLAB // 2 - 1)
            def _():
                _swait(0)
                _gissue(es, kk + 2, 0)

            _scale(1, vs, kk + 1)
            _sissue(es, kk + 1, 1)
            return carry

        lax.fori_loop(0, SLAB // 2, _pair, 0)

    pltpu.async_copy(edges.at[pl.ds(s * CPW, SLAB)], eslabA, si)
    pltpu.async_copy(vals.at[pl.ds(s * CPW, SLAB)], vslabA, si)

    def _slabpair(u, carry):
        _slab_body(2 * u, eslabA, vslabA, eslabB, vslabB)
        _slab_body(2 * u + 1, eslabB, vslabB, eslabA, vslabA)
        return carry

    lax.fori_loop(0, NSLB // 2, _slabpair, 0)
    _swait(0)
    _swait(1)
    plsc.subcore_barrier()

    nw_blocks = RPT // RB
    pltpu.async_copy(acc.at[pl.ds(s * RPT, RB)], rows[0], sg[0])
    for kb in range(nw_blocks):
        rbw = kb % 2
        r0 = s * RPT + kb * RB
        pltpu.make_async_copy(acc.at[pl.ds(r0, RB)], rows[rbw], sg[rbw]).wait()
        if kb + 1 < nw_blocks:
            if kb >= 1:
                pltpu.make_async_copy(rows[1 - rbw],
                                      out_st.at[c].at[pl.ds(s * RPT, RB)],
                                      ss[1 - rbw]).wait()
            pltpu.async_copy(acc.at[pl.ds(r0 + RB, RB)], rows[1 - rbw],
                             sg[1 - rbw])

        def _rbody(r, carry):
            for j in range(F // 16):
                rows[rbw][r, pl.ds(j * 16, 16)] = jnp.maximum(
                    rows[rbw][r, pl.ds(j * 16, 16)], 0.0)
            return carry

        lax.fori_loop(0, RB, _rbody, 0)
        pltpu.async_copy(rows[rbw], out_st.at[c].at[pl.ds(r0, RB)], ss[rbw])
    for rbw in range(2):
        pltpu.make_async_copy(rows[rbw], out_st.at[c].at[pl.ds(s * RPT, RB)],
                              ss[rbw]).wait()


def _prep_edges(idx2, val):
    padc = NCHP - NCHT
    dst2 = jnp.pad(idx2[0].reshape(NCHT, CH), ((0, padc), (0, 0)),
                   constant_values=NP - 1)
    src2 = jnp.pad(idx2[1].reshape(NCHT, CH), ((0, padc), (0, 0)))
    val2 = jnp.pad(val.reshape(NCHT, CH), ((0, padc), (0, 0)))
    return jnp.stack([dst2, src2], axis=1), val2


def kernel(adj_indices, adj_values, adj2_indices, adj2_values,
           emb_node, emb_attri, W_trans1, W_gc1, W_gc2):
    x0 = jnp.concatenate([emb_node, emb_attri], axis=0)
    x1, h_st = _dense(x0, W_trans1, W_gc1, W_gc2)
    e1, v1 = _prep_edges(adj_indices, adj_values)
    e2, v2 = _prep_edges(adj2_indices, adj2_values)
    edges_st = jnp.stack([e1, e2])
    vals_st = jnp.stack([v1, v2])
    out = _sc_spmm(h_st, edges_st, vals_st)
    return (x1, out[0, :N_TOT], out[1, :N_TOT])
